# Initial kernel scaffold; baseline (speedup 1.0000x reference)
import functools

import jax
import jax.numpy as jnp
import numpy as np
from jax import lax
from jax.experimental import pallas as pl
from jax.experimental.pallas import tpu as pltpu
from jax.experimental.pallas import tpu_sc as plsc

_LEVELS = 16
_TABLE_SIZE = 524288
_FEAT = 2
_DIM = 3
_N = 262144
_MASK = _TABLE_SIZE - 1

_P1 = np.uint32(2654435761)
_P2 = np.uint32(805459861)

_NC = 2
_NS = 16
_NW = _NC * _NS
_PPW = _N // _NW
_C = 512
_NCHUNK = _PPW // _C


def _res_values():
    b = np.exp((np.log(512.0) - np.log(16.0)) / (_LEVELS - 1))
    return np.floor(16.0 * (b ** np.arange(_LEVELS))).astype(np.float32)


_RES = _res_values()


def _body(x_hbm, tab_hbm, out_hbm, xbuf, fbuf, idxbuf, featbuf, obuf, sem):
    wid = lax.axis_index("s") * _NC + lax.axis_index("c")
    base = wid * _PPW
    iota = lax.iota(jnp.int32, 16)
    zeros = jnp.zeros(16, jnp.int32)
    ones = jnp.ones(16, jnp.int32)

    def chunk_body(ci, carry):
        cb = base + ci * _C
        pltpu.sync_copy(x_hbm.at[:, pl.ds(cb, _C)], xbuf)

        for l in range(_LEVELS):
            res_l = float(_RES[l])
            lvl_base = np.int32(l * _TABLE_SIZE)

            def pass_a(j, c2, res_l=res_l, lvl_base=lvl_base):
                off = j * 16
                sx = xbuf[0, pl.ds(off, 16)] * res_l
                sy = xbuf[1, pl.ds(off, 16)] * res_l
                sz = xbuf[2, pl.ds(off, 16)] * res_l
                cx = sx.astype(jnp.int32)
                cy = sy.astype(jnp.int32)
                cz = sz.astype(jnp.int32)
                fbuf[0, pl.ds(off, 16)] = sx - cx.astype(jnp.float32)
                fbuf[1, pl.ds(off, 16)] = sy - cy.astype(jnp.float32)
                fbuf[2, pl.ds(off, 16)] = sz - cz.astype(jnp.float32)
                ux = cx.astype(jnp.uint32)
                uy = cy.astype(jnp.uint32)
                uz = cz.astype(jnp.uint32)
                hx = (ux, ux + np.uint32(1))
                hy = (uy * _P1, (uy + np.uint32(1)) * _P1)
                hz = (uz * _P2, (uz + np.uint32(1)) * _P2)
                for c in range(8):
                    h = hx[(c >> 2) & 1] ^ hy[(c >> 1) & 1] ^ hz[c & 1]
                    idx = (h & np.uint32(_MASK)).astype(jnp.int32) + lvl_base
                    idxbuf[pl.ds(c * _C + off, 16)] = idx
                return c2

            lax.fori_loop(0, _C // 16, pass_a, 0)

            pltpu.async_copy(tab_hbm.at[idxbuf], featbuf, sem).wait()

            def pass_b(j, c2, l=l):
                off = j * 16
                f0 = fbuf[0, pl.ds(off, 16)]
                f1 = fbuf[1, pl.ds(off, 16)]
                f2 = fbuf[2, pl.ds(off, 16)]
                wx = (1.0 - f0, f0)
                wy = (1.0 - f1, f1)
                wz = (1.0 - f2, f2)
                rows0 = off + iota
                acc0 = jnp.zeros(16, jnp.float32)
                acc1 = jnp.zeros(16, jnp.float32)
                for c in range(8):
                    w = wx[(c >> 2) & 1] * wy[(c >> 1) & 1] * wz[c & 1]
                    rows = rows0 + (c * _C)
                    a0 = plsc.load_gather(featbuf, [rows, zeros])
                    a1 = plsc.load_gather(featbuf, [rows, ones])
                    acc0 = acc0 + a0 * w
                    acc1 = acc1 + a1 * w
                col0 = jnp.full((16,), 2 * l, jnp.int32)
                plsc.store_scatter(obuf, [rows0, col0], acc0)
                plsc.store_scatter(obuf, [rows0, col0 + 1], acc1)
                return c2

            lax.fori_loop(0, _C // 16, pass_b, 0)

        pltpu.sync_copy(obuf, out_hbm.at[pl.ds(cb, _C)])
        return carry

    lax.fori_loop(0, _NCHUNK, chunk_body, 0)


@jax.jit
def _encode(xt, tab):
    mesh = plsc.VectorSubcoreMesh(core_axis_name="c", subcore_axis_name="s")
    return pl.kernel(
        _body,
        out_type=jax.ShapeDtypeStruct((_N, _LEVELS * _FEAT), jnp.float32),
        mesh=mesh,
        scratch_types=[
            pltpu.VMEM((_DIM, _C), jnp.float32),
            pltpu.VMEM((_DIM, _C), jnp.float32),
            pltpu.VMEM((8 * _C,), jnp.int32),
            pltpu.VMEM((8 * _C, _FEAT), jnp.float32),
            pltpu.VMEM((_C, _LEVELS * _FEAT), jnp.float32),
            pltpu.SemaphoreType.DMA,
        ],
    )(xt, tab)


def kernel(x, hash_tables):
    xt = x.T
    tab = hash_tables.reshape(_LEVELS * _TABLE_SIZE, _FEAT)
    return _encode(xt, tab)


# speedup vs baseline: 12.6156x; 12.6156x over previous
import jax
import jax.numpy as jnp
import numpy as np
from jax import lax
from jax.experimental import pallas as pl
from jax.experimental.pallas import tpu as pltpu
from jax.experimental.pallas import tpu_sc as plsc

_LEVELS = 16
_TABLE_SIZE = 524288
_N = 262144
_MASK = _TABLE_SIZE - 1

_P1 = np.uint32(2654435761)
_P2 = np.uint32(805459861)

_NC = 2
_NS = 16
_PPW = _N // (_NC * _NS)
_C = 512
_NCHUNK = _PPW // _C
_PP = 32
_G = _PP * 8
_NPIECE = _C // _PP
_RPL = _TABLE_SIZE // 64


def _res_values():
    b = np.exp((np.log(512.0) - np.log(16.0)) / (_LEVELS - 1))
    return np.floor(16.0 * (b ** np.arange(_LEVELS))).astype(np.float32)


def _body(x_hbm, tab_hbm, res_hbm, out_hbm, xbuf, fbuf, idxbuf, lobuf,
          featbuf, ob, resbuf, sem_g):
    sid = lax.axis_index("s")
    wid = lax.axis_index("c") * _NS + sid
    base = wid * _PPW
    iota = lax.iota(jnp.int32, 16)
    pltpu.sync_copy(x_hbm.at[:, pl.ds(base, _PPW)], xbuf)
    pltpu.sync_copy(res_hbm, resbuf)
    res_vec = resbuf[pl.ds(0, 16)]

    def level_body(l, carry0):
        res_l = jnp.sum(jnp.where(iota == l, res_vec, 0.0))
        lbase = l * _RPL

        def chunk_body(ci, carry):
            cb = ci * _C

            def pass_a(j, c2):
                off = cb + j * 16
                sx = xbuf[0, pl.ds(off, 16)] * res_l
                sy = xbuf[1, pl.ds(off, 16)] * res_l
                sz = xbuf[2, pl.ds(off, 16)] * res_l
                cx = sx.astype(jnp.int32)
                cy = sy.astype(jnp.int32)
                cz = sz.astype(jnp.int32)
                o = j * 16
                fbuf[0, pl.ds(o, 16)] = sx - cx.astype(jnp.float32)
                fbuf[1, pl.ds(o, 16)] = sy - cy.astype(jnp.float32)
                fbuf[2, pl.ds(o, 16)] = sz - cz.astype(jnp.float32)
                ux = cx.astype(jnp.uint32)
                uy = cy.astype(jnp.uint32)
                uz = cz.astype(jnp.uint32)
                hx = (ux, ux + np.uint32(1))
                hy = (uy * _P1, (uy + np.uint32(1)) * _P1)
                hz = (uz * _P2, (uz + np.uint32(1)) * _P2)
                pbase = (j // 2) * _G + (j % 2) * 16
                for c in range(8):
                    h = hx[(c >> 2) & 1] ^ hy[(c >> 1) & 1] ^ hz[c & 1]
                    hm = (h & np.uint32(_MASK)).astype(jnp.int32)
                    idxbuf[pl.ds(pbase + c * _PP, 16)] = (
                        lbase + (hm >> 6))
                    lobuf[pl.ds(pbase + c * _PP, 16)] = (
                        (hm & 63) * 2)
                return c2

            lax.fori_loop(0, _C // 16, pass_a, 0)

            def piece_body(g, c2):
                pltpu.async_copy(
                    tab_hbm.at[idxbuf.at[pl.ds(g * _G, _G)]], featbuf, sem_g
                ).wait()

                def pass_b(b, c3):
                    o = g * _PP + b * 16
                    f0 = fbuf[0, pl.ds(o, 16)]
                    f1 = fbuf[1, pl.ds(o, 16)]
                    f2 = fbuf[2, pl.ds(o, 16)]
                    wx = (1.0 - f0, f0)
                    wy = (1.0 - f1, f1)
                    wz = (1.0 - f2, f2)
                    acc0 = jnp.zeros(16, jnp.float32)
                    acc1 = jnp.zeros(16, jnp.float32)
                    rb = b * 16 + iota
                    sb = b * 16
                    for c in range(8):
                        w = wx[(c >> 2) & 1] * wy[(c >> 1) & 1] * wz[c & 1]
                        lo = lobuf[pl.ds(g * _G + c * _PP + sb, 16)]
                        rows = rb + c * _PP
                        a0 = plsc.load_gather(featbuf, [rows, lo])
                        a1 = plsc.load_gather(featbuf, [rows, lo + 1])
                        acc0 = acc0 + a0 * w
                        acc1 = acc1 + a1 * w
                    ob[0, pl.ds(cb + o, 16)] = acc0
                    ob[1, pl.ds(cb + o, 16)] = acc1
                    return c3

                lax.fori_loop(0, _PP // 16, pass_b, 0)
                return c2

            lax.fori_loop(0, _NPIECE, piece_body, 0)
            return carry

        lax.fori_loop(0, _NCHUNK, chunk_body, 0)

        pltpu.sync_copy(ob, out_hbm.at[pl.ds(2 * l, 2), pl.ds(base, _PPW)])
        return carry0

    lax.fori_loop(0, _LEVELS, level_body, 0)


@jax.jit
def _encode(xt, tab, resv):
    mesh = plsc.VectorSubcoreMesh(core_axis_name="c", subcore_axis_name="s")
    return pl.kernel(
        _body,
        out_type=jax.ShapeDtypeStruct((_LEVELS * 2, _N), jnp.float32),
        mesh=mesh,
        scratch_types=[
            pltpu.VMEM((3, _PPW), jnp.float32),
            pltpu.VMEM((3, _C), jnp.float32),
            pltpu.VMEM((8 * _C,), jnp.int32),
            pltpu.VMEM((8 * _C,), jnp.int32),
            pltpu.VMEM((_G, 128), jnp.float32),
            pltpu.VMEM((2, _PPW), jnp.float32),
            pltpu.VMEM((16,), jnp.float32),
            pltpu.SemaphoreType.DMA,
        ],
        compiler_params=pltpu.CompilerParams(needs_layout_passes=False),
    )(xt, tab, resv)


def kernel(x, hash_tables):
    tab = hash_tables.reshape(_LEVELS * _RPL, 128)
    out = _encode(x.T, tab, jnp.asarray(_res_values()))
    return out.T


# speedup vs baseline: 13.2754x; 1.0523x over previous
import jax
import jax.numpy as jnp
import numpy as np
from jax import lax
from jax.experimental import pallas as pl
from jax.experimental.pallas import tpu as pltpu
from jax.experimental.pallas import tpu_sc as plsc

_LEVELS = 16
_TABLE_SIZE = 524288
_N = 262144
_MASK = _TABLE_SIZE - 1

_P1 = np.uint32(2654435761)
_P2 = np.uint32(805459861)

_NC = 2
_NS = 16
_PPW = _N // (_NC * _NS)
_C = 512
_NCHUNK = _PPW // _C
_PP = 32
_G = _PP * 8
_NPIECE = _C // _PP
_RPL = _TABLE_SIZE // 64


def _res_values():
    b = np.exp((np.log(512.0) - np.log(16.0)) / (_LEVELS - 1))
    return np.floor(16.0 * (b ** np.arange(_LEVELS))).astype(np.float32)


def _body(x_hbm, tab_hbm, res_hbm, out_hbm, xbuf, fbuf, idxbuf, lobuf,
          feat_a, feat_b, ob, resbuf, sem_a, sem_b):
    sid = lax.axis_index("s")
    wid = lax.axis_index("c") * _NS + sid
    base = wid * _PPW
    iota = lax.iota(jnp.int32, 16)
    pltpu.sync_copy(x_hbm.at[:, pl.ds(base, _PPW)], xbuf)
    pltpu.sync_copy(res_hbm, resbuf)
    res_vec = resbuf[pl.ds(0, 16)]

    def level_body(l, carry0):
        res_l = jnp.sum(jnp.where(iota == l, res_vec, 0.0))
        lbase = l * _RPL

        def chunk_body(ci, carry):
            cb = ci * _C

            def pass_a(j, c2):
                off = cb + j * 16
                sx = xbuf[0, pl.ds(off, 16)] * res_l
                sy = xbuf[1, pl.ds(off, 16)] * res_l
                sz = xbuf[2, pl.ds(off, 16)] * res_l
                cx = sx.astype(jnp.int32)
                cy = sy.astype(jnp.int32)
                cz = sz.astype(jnp.int32)
                o = j * 16
                fbuf[0, pl.ds(o, 16)] = sx - cx.astype(jnp.float32)
                fbuf[1, pl.ds(o, 16)] = sy - cy.astype(jnp.float32)
                fbuf[2, pl.ds(o, 16)] = sz - cz.astype(jnp.float32)
                ux = cx.astype(jnp.uint32)
                uy = cy.astype(jnp.uint32)
                uz = cz.astype(jnp.uint32)
                hx = (ux, ux + np.uint32(1))
                hy = (uy * _P1, (uy + np.uint32(1)) * _P1)
                hz = (uz * _P2, (uz + np.uint32(1)) * _P2)
                pbase = (j // 2) * _G + (j % 2) * 16
                for c in range(8):
                    h = hx[(c >> 2) & 1] ^ hy[(c >> 1) & 1] ^ hz[c & 1]
                    hm = (h & np.uint32(_MASK)).astype(jnp.int32)
                    idxbuf[pl.ds(pbase + c * _PP, 16)] = (
                        lbase + (hm >> 6))
                    lobuf[pl.ds(pbase + c * _PP, 16)] = (
                        (hm & 63) * 2)
                return c2

            lax.fori_loop(0, _C // 16, pass_a, 0)

            def fire(g, fb, sem):
                pltpu.async_copy(
                    tab_hbm.at[idxbuf.at[pl.ds(g * _G, _G)]], fb, sem)

            def drain(fb, sem):
                pltpu.make_async_copy(
                    tab_hbm.at[idxbuf.at[pl.ds(0, _G)]], fb, sem).wait()

            def pass_b_piece(g, fb):
                def pass_b(b, c3):
                    o = g * _PP + b * 16
                    f0 = fbuf[0, pl.ds(o, 16)]
                    f1 = fbuf[1, pl.ds(o, 16)]
                    f2 = fbuf[2, pl.ds(o, 16)]
                    wx = (1.0 - f0, f0)
                    wy = (1.0 - f1, f1)
                    wz = (1.0 - f2, f2)
                    acc0 = jnp.zeros(16, jnp.float32)
                    acc1 = jnp.zeros(16, jnp.float32)
                    rb = b * 16 + iota
                    sb = b * 16
                    for c in range(8):
                        w = wx[(c >> 2) & 1] * wy[(c >> 1) & 1] * wz[c & 1]
                        lo = lobuf[pl.ds(g * _G + c * _PP + sb, 16)]
                        rows = rb + c * _PP
                        a0 = plsc.load_gather(fb, [rows, lo])
                        a1 = plsc.load_gather(fb, [rows, lo + 1])
                        acc0 = acc0 + a0 * w
                        acc1 = acc1 + a1 * w
                    ob[0, pl.ds(cb + o, 16)] = acc0
                    ob[1, pl.ds(cb + o, 16)] = acc1
                    return c3

                lax.fori_loop(0, _PP // 16, pass_b, 0)

            fire(0, feat_a, sem_a)
            fire(1, feat_b, sem_b)

            def pair_body(gg, c2):
                g0 = gg * 2
                drain(feat_a, sem_a)
                pass_b_piece(g0, feat_a)
                fire(g0 + 2, feat_a, sem_a)
                drain(feat_b, sem_b)
                pass_b_piece(g0 + 1, feat_b)
                fire(g0 + 3, feat_b, sem_b)
                return c2

            lax.fori_loop(0, _NPIECE // 2 - 1, pair_body, 0)
            drain(feat_a, sem_a)
            pass_b_piece(_NPIECE - 2, feat_a)
            drain(feat_b, sem_b)
            pass_b_piece(_NPIECE - 1, feat_b)
            return carry

        lax.fori_loop(0, _NCHUNK, chunk_body, 0)

        pltpu.sync_copy(ob, out_hbm.at[pl.ds(2 * l, 2), pl.ds(base, _PPW)])
        return carry0

    lax.fori_loop(0, _LEVELS, level_body, 0)


@jax.jit
def _encode(xt, tab, resv):
    mesh = plsc.VectorSubcoreMesh(core_axis_name="c", subcore_axis_name="s")
    return pl.kernel(
        _body,
        out_type=jax.ShapeDtypeStruct((_LEVELS * 2, _N), jnp.float32),
        mesh=mesh,
        scratch_types=[
            pltpu.VMEM((3, _PPW), jnp.float32),
            pltpu.VMEM((3, _C), jnp.float32),
            pltpu.VMEM((8 * _C,), jnp.int32),
            pltpu.VMEM((8 * _C,), jnp.int32),
            pltpu.VMEM((_G, 128), jnp.float32),
            pltpu.VMEM((_G, 128), jnp.float32),
            pltpu.VMEM((2, _PPW), jnp.float32),
            pltpu.VMEM((16,), jnp.float32),
            pltpu.SemaphoreType.DMA,
            pltpu.SemaphoreType.DMA,
        ],
        compiler_params=pltpu.CompilerParams(needs_layout_passes=False),
    )(xt, tab, resv)


def kernel(x, hash_tables):
    tab = hash_tables.reshape(_LEVELS * _RPL, 128)
    out = _encode(x.T, tab, jnp.asarray(_res_values()))
    return out.T
